# baseline (speedup 1.0000x reference)
import functools

import jax
import jax.numpy as jnp
from jax import lax
from jax.experimental import pallas as pl
from jax.experimental.pallas import tpu as pltpu
from jax.experimental.pallas import tpu_sc as plsc

NUM_T = 1000
IMG_SHAPE = (3, 64, 64)
BATCH = 1024
FEAT = 3 * 64 * 64

_NC = 2
_NS = 16
_L = 16
_NW = _NC * _NS
_PER_W = BATCH // _NW
_TBL_PAD = 1024


def _schedule_tables():
    scale = 1000.0 / NUM_T
    beta = jnp.linspace(scale * 0.0001, scale * 0.02, NUM_T, dtype=jnp.float32)
    alpha_cum = jnp.cumprod(1.0 - beta, axis=0)
    sqrt_ac = jnp.sqrt(alpha_cum)
    sqrt_omac = jnp.sqrt(1.0 - alpha_cum)
    pad = _TBL_PAD - NUM_T
    return (jnp.pad(sqrt_ac, (0, pad)), jnp.pad(sqrt_omac, (0, pad)))


def _sc_gather_body(ts_hbm, ac_hbm, om_hbm, coef_hbm, std_hbm,
                    ac_v, om_v, idx_v, coef_v, std_v):
    wid = lax.axis_index("s") * _NC + lax.axis_index("c")
    base = wid * _PER_W
    pltpu.sync_copy(ac_hbm, ac_v)
    pltpu.sync_copy(om_hbm, om_v)
    pltpu.sync_copy(ts_hbm.at[pl.ds(base, _PER_W)], idx_v)
    for j in range(_PER_W // _L):
        idx = idx_v[pl.ds(j * _L, _L)]
        coef_v[pl.ds(j * _L, _L)] = plsc.load_gather(ac_v, [idx])
        std_v[pl.ds(j * _L, _L)] = plsc.load_gather(om_v, [idx])
    pltpu.sync_copy(coef_v, coef_hbm.at[pl.ds(base, _PER_W)])
    pltpu.sync_copy(std_v, std_hbm.at[pl.ds(base, _PER_W)])


@functools.lru_cache(maxsize=None)
def _sc_gather_fn():
    return pl.kernel(
        _sc_gather_body,
        out_type=(
            jax.ShapeDtypeStruct((BATCH,), jnp.float32),
            jax.ShapeDtypeStruct((BATCH,), jnp.float32),
        ),
        mesh=plsc.VectorSubcoreMesh(core_axis_name="c", subcore_axis_name="s"),
        compiler_params=pltpu.CompilerParams(needs_layout_passes=False),
        scratch_types=[
            pltpu.VMEM((_TBL_PAD,), jnp.float32),
            pltpu.VMEM((_TBL_PAD,), jnp.float32),
            pltpu.VMEM((_PER_W,), jnp.int32),
            pltpu.VMEM((_PER_W,), jnp.float32),
            pltpu.VMEM((_PER_W,), jnp.float32),
        ],
    )


_RF = 768


def _scale_body(coef_ref, std_ref, x_ref, e_ref, out_ref, eout_ref):
    e = e_ref[...]
    out_ref[...] = coef_ref[...] * x_ref[...] + std_ref[...] * e
    eout_ref[...] = e


def _tc_scale(coef, std, xT, eT):
    grid = (FEAT // _RF,)
    blk = pl.BlockSpec((_RF, BATCH), lambda i: (i, 0))
    row = pl.BlockSpec((1, BATCH), lambda i: (0, 0))
    return pl.pallas_call(
        _scale_body,
        grid=grid,
        in_specs=[row, row, blk, blk],
        out_specs=(blk, blk),
        out_shape=(jax.ShapeDtypeStruct((FEAT, BATCH), jnp.float32),
                   jax.ShapeDtypeStruct((FEAT, BATCH), jnp.float32)),
    )(coef, std, xT, eT)


def kernel(x0, timesteps, eps):
    sqrt_ac, sqrt_omac = _schedule_tables()
    coef, std = _sc_gather_fn()(timesteps.astype(jnp.int32), sqrt_ac, sqrt_omac)
    xT = x0.transpose(1, 2, 3, 0).reshape(FEAT, BATCH)
    eT = eps.transpose(1, 2, 3, 0).reshape(FEAT, BATCH)
    outT, eoutT = _tc_scale(coef.reshape(1, BATCH), std.reshape(1, BATCH),
                            xT, eT)
    sample = outT.reshape(IMG_SHAPE + (BATCH,)).transpose(3, 0, 1, 2)
    eps_out = eoutT.reshape(IMG_SHAPE + (BATCH,)).transpose(3, 0, 1, 2)
    return (sample, eps_out)


# speedup vs baseline: 1.0261x; 1.0261x over previous
import functools

import jax
import jax.numpy as jnp
from jax import lax
from jax.experimental import pallas as pl
from jax.experimental.pallas import tpu as pltpu
from jax.experimental.pallas import tpu_sc as plsc

NUM_T = 1000
IMG_SHAPE = (3, 64, 64)
BATCH = 1024
FEAT = 3 * 64 * 64

_NC = 2
_NS = 16
_L = 16
_NW = _NC * _NS
_PER_W = BATCH // _NW
_TBL_PAD = 1024


def _schedule_tables():
    scale = 1000.0 / NUM_T
    beta = jnp.linspace(scale * 0.0001, scale * 0.02, NUM_T, dtype=jnp.float32)
    alpha_cum = jnp.cumprod(1.0 - beta, axis=0)
    sqrt_ac = jnp.sqrt(alpha_cum)
    sqrt_omac = jnp.sqrt(1.0 - alpha_cum)
    pad = _TBL_PAD - NUM_T
    return jnp.concatenate(
        [jnp.pad(sqrt_ac, (0, pad)), jnp.pad(sqrt_omac, (0, pad))])


def _sc_gather_body(ts_hbm, tbl_hbm, out_hbm, tbl_v, idx_v, res_v,
                    sem_t, sem_i, sem_c, sem_s):
    wid = lax.axis_index("s") * _NC + lax.axis_index("c")
    base = wid * _PER_W
    cp_t = pltpu.make_async_copy(tbl_hbm, tbl_v, sem_t)
    cp_i = pltpu.make_async_copy(ts_hbm.at[pl.ds(base, _PER_W)], idx_v, sem_i)
    cp_t.start()
    cp_i.start()
    cp_t.wait()
    cp_i.wait()
    for j in range(_PER_W // _L):
        idx = idx_v[pl.ds(j * _L, _L)]
        res_v[pl.ds(j * _L, _L)] = plsc.load_gather(tbl_v, [idx])
        res_v[pl.ds(_PER_W + j * _L, _L)] = plsc.load_gather(
            tbl_v, [idx + _TBL_PAD])
    cp_c = pltpu.make_async_copy(res_v.at[pl.ds(0, _PER_W)],
                                 out_hbm.at[pl.ds(base, _PER_W)], sem_c)
    cp_s = pltpu.make_async_copy(res_v.at[pl.ds(_PER_W, _PER_W)],
                                 out_hbm.at[pl.ds(_TBL_PAD + base, _PER_W)],
                                 sem_s)
    cp_c.start()
    cp_s.start()
    cp_c.wait()
    cp_s.wait()


@functools.lru_cache(maxsize=None)
def _sc_gather_fn():
    return pl.kernel(
        _sc_gather_body,
        out_type=jax.ShapeDtypeStruct((2 * _TBL_PAD,), jnp.float32),
        mesh=plsc.VectorSubcoreMesh(core_axis_name="c", subcore_axis_name="s"),
        compiler_params=pltpu.CompilerParams(needs_layout_passes=False),
        scratch_types=[
            pltpu.VMEM((2 * _TBL_PAD,), jnp.float32),
            pltpu.VMEM((_PER_W,), jnp.int32),
            pltpu.VMEM((2 * _PER_W,), jnp.float32),
            pltpu.SemaphoreType.DMA,
            pltpu.SemaphoreType.DMA,
            pltpu.SemaphoreType.DMA,
            pltpu.SemaphoreType.DMA,
        ],
    )


_RF = 1536


def _scale_body(coef_ref, std_ref, x_ref, e_ref, out_ref, eout_ref):
    e = e_ref[...]
    out_ref[...] = coef_ref[...] * x_ref[...] + std_ref[...] * e
    eout_ref[...] = e


def _tc_scale(coef, std, xT, eT):
    grid = (FEAT // _RF,)
    blk = pl.BlockSpec((_RF, BATCH), lambda i: (i, 0))
    row = pl.BlockSpec((1, BATCH), lambda i: (0, 0))
    return pl.pallas_call(
        _scale_body,
        grid=grid,
        in_specs=[row, row, blk, blk],
        out_specs=(blk, blk),
        out_shape=(jax.ShapeDtypeStruct((FEAT, BATCH), jnp.float32),
                   jax.ShapeDtypeStruct((FEAT, BATCH), jnp.float32)),
    )(coef, std, xT, eT)


def kernel(x0, timesteps, eps):
    tbl = _schedule_tables()
    cs = _sc_gather_fn()(timesteps.astype(jnp.int32), tbl)
    coef, std = cs[:BATCH], cs[_TBL_PAD:_TBL_PAD + BATCH]
    xT = x0.transpose(1, 2, 3, 0).reshape(FEAT, BATCH)
    eT = eps.transpose(1, 2, 3, 0).reshape(FEAT, BATCH)
    outT, eoutT = _tc_scale(coef.reshape(1, BATCH), std.reshape(1, BATCH),
                            xT, eT)
    sample = outT.reshape(IMG_SHAPE + (BATCH,)).transpose(3, 0, 1, 2)
    eps_out = eoutT.reshape(IMG_SHAPE + (BATCH,)).transpose(3, 0, 1, 2)
    return (sample, eps_out)


# speedup vs baseline: 1.0460x; 1.0194x over previous
import functools

import jax
import jax.numpy as jnp
from jax import lax
from jax.experimental import pallas as pl
from jax.experimental.pallas import tpu as pltpu
from jax.experimental.pallas import tpu_sc as plsc

NUM_T = 1000
IMG_SHAPE = (3, 64, 64)
BATCH = 1024
FEAT = 3 * 64 * 64

_NC = 1
_NS = 16
_L = 16
_NW = _NC * _NS
_PER_W = BATCH // _NW
_TBL_PAD = 1024


def _schedule_tables():
    scale = 1000.0 / NUM_T
    beta = jnp.linspace(scale * 0.0001, scale * 0.02, NUM_T, dtype=jnp.float32)
    alpha_cum = jnp.cumprod(1.0 - beta, axis=0)
    sqrt_ac = jnp.sqrt(alpha_cum)
    sqrt_omac = jnp.sqrt(1.0 - alpha_cum)
    pad = _TBL_PAD - NUM_T
    return jnp.concatenate(
        [jnp.pad(sqrt_ac, (0, pad)), jnp.pad(sqrt_omac, (0, pad))])


def _sc_gather_body(ts_hbm, tbl_hbm, out_hbm, tbl_v, idx_v, res_v,
                    sem_t, sem_i, sem_c, sem_s):
    wid = lax.axis_index("s") * _NC + lax.axis_index("c")
    base = wid * _PER_W
    cp_t = pltpu.make_async_copy(tbl_hbm, tbl_v, sem_t)
    cp_i = pltpu.make_async_copy(ts_hbm.at[pl.ds(base, _PER_W)], idx_v, sem_i)
    cp_t.start()
    cp_i.start()
    cp_t.wait()
    cp_i.wait()
    for j in range(_PER_W // _L):
        idx = idx_v[pl.ds(j * _L, _L)]
        res_v[pl.ds(j * _L, _L)] = plsc.load_gather(tbl_v, [idx])
        res_v[pl.ds(_PER_W + j * _L, _L)] = plsc.load_gather(
            tbl_v, [idx + _TBL_PAD])
    cp_c = pltpu.make_async_copy(res_v.at[pl.ds(0, _PER_W)],
                                 out_hbm.at[pl.ds(base, _PER_W)], sem_c)
    cp_s = pltpu.make_async_copy(res_v.at[pl.ds(_PER_W, _PER_W)],
                                 out_hbm.at[pl.ds(_TBL_PAD + base, _PER_W)],
                                 sem_s)
    cp_c.start()
    cp_s.start()
    cp_c.wait()
    cp_s.wait()


@functools.lru_cache(maxsize=None)
def _sc_gather_fn():
    return pl.kernel(
        _sc_gather_body,
        out_type=jax.ShapeDtypeStruct((2 * _TBL_PAD,), jnp.float32),
        mesh=plsc.VectorSubcoreMesh(core_axis_name="c", subcore_axis_name="s", num_cores=1),
        compiler_params=pltpu.CompilerParams(needs_layout_passes=False),
        scratch_types=[
            pltpu.VMEM((2 * _TBL_PAD,), jnp.float32),
            pltpu.VMEM((_PER_W,), jnp.int32),
            pltpu.VMEM((2 * _PER_W,), jnp.float32),
            pltpu.SemaphoreType.DMA,
            pltpu.SemaphoreType.DMA,
            pltpu.SemaphoreType.DMA,
            pltpu.SemaphoreType.DMA,
        ],
    )


_RF = 1536


def _scale_body(coef_ref, std_ref, x_ref, e_ref, out_ref, eout_ref):
    e = e_ref[...]
    out_ref[...] = coef_ref[...] * x_ref[...] + std_ref[...] * e
    eout_ref[...] = e


def _tc_scale(coef, std, xT, eT):
    grid = (FEAT // _RF,)
    blk = pl.BlockSpec((_RF, BATCH), lambda i: (i, 0))
    row = pl.BlockSpec((1, BATCH), lambda i: (0, 0))
    return pl.pallas_call(
        _scale_body,
        grid=grid,
        in_specs=[row, row, blk, blk],
        out_specs=(blk, blk),
        out_shape=(jax.ShapeDtypeStruct((FEAT, BATCH), jnp.float32),
                   jax.ShapeDtypeStruct((FEAT, BATCH), jnp.float32)),
    )(coef, std, xT, eT)


def kernel(x0, timesteps, eps):
    tbl = _schedule_tables()
    cs = _sc_gather_fn()(timesteps.astype(jnp.int32), tbl)
    coef, std = cs[:BATCH], cs[_TBL_PAD:_TBL_PAD + BATCH]
    xT = x0.transpose(1, 2, 3, 0).reshape(FEAT, BATCH)
    eT = eps.transpose(1, 2, 3, 0).reshape(FEAT, BATCH)
    outT, eoutT = _tc_scale(coef.reshape(1, BATCH), std.reshape(1, BATCH),
                            xT, eT)
    sample = outT.reshape(IMG_SHAPE + (BATCH,)).transpose(3, 0, 1, 2)
    eps_out = eoutT.reshape(IMG_SHAPE + (BATCH,)).transpose(3, 0, 1, 2)
    return (sample, eps_out)


# speedup vs baseline: 1.0489x; 1.0028x over previous
import functools

import jax
import jax.numpy as jnp
from jax import lax
from jax.experimental import pallas as pl
from jax.experimental.pallas import tpu as pltpu
from jax.experimental.pallas import tpu_sc as plsc

NUM_T = 1000
IMG_SHAPE = (3, 64, 64)
BATCH = 1024
FEAT = 3 * 64 * 64

_NC = 1
_NS = 16
_L = 16
_NW = _NC * _NS
_PER_W = BATCH // _NW
_TBL_PAD = 1024


def _schedule_tables():
    scale = 1000.0 / NUM_T
    beta = jnp.linspace(scale * 0.0001, scale * 0.02, NUM_T, dtype=jnp.float32)
    alpha_cum = jnp.cumprod(1.0 - beta, axis=0)
    sqrt_ac = jnp.sqrt(alpha_cum)
    sqrt_omac = jnp.sqrt(1.0 - alpha_cum)
    pad = _TBL_PAD - NUM_T
    return jnp.concatenate(
        [jnp.pad(sqrt_ac, (0, pad)), jnp.pad(sqrt_omac, (0, pad))])


def _sc_gather_body(ts_hbm, tbl_hbm, out_hbm, tbl_v, idx_v, res_v,
                    sem_t, sem_i, sem_c, sem_s):
    wid = lax.axis_index("s") * _NC + lax.axis_index("c")
    base = wid * _PER_W
    cp_t = pltpu.make_async_copy(tbl_hbm, tbl_v, sem_t)
    cp_i = pltpu.make_async_copy(ts_hbm.at[pl.ds(base, _PER_W)], idx_v, sem_i)
    cp_t.start()
    cp_i.start()
    cp_t.wait()
    cp_i.wait()
    for j in range(_PER_W // _L):
        idx = idx_v[pl.ds(j * _L, _L)]
        res_v[pl.ds(j * _L, _L)] = plsc.load_gather(tbl_v, [idx])
        res_v[pl.ds(_PER_W + j * _L, _L)] = plsc.load_gather(
            tbl_v, [idx + _TBL_PAD])
    cp_c = pltpu.make_async_copy(res_v.at[pl.ds(0, _PER_W)],
                                 out_hbm.at[pl.ds(base, _PER_W)], sem_c)
    cp_s = pltpu.make_async_copy(res_v.at[pl.ds(_PER_W, _PER_W)],
                                 out_hbm.at[pl.ds(_TBL_PAD + base, _PER_W)],
                                 sem_s)
    cp_c.start()
    cp_s.start()
    cp_c.wait()
    cp_s.wait()


@functools.lru_cache(maxsize=None)
def _sc_gather_fn():
    return pl.kernel(
        _sc_gather_body,
        out_type=jax.ShapeDtypeStruct((2 * _TBL_PAD,), jnp.float32),
        mesh=plsc.VectorSubcoreMesh(core_axis_name="c", subcore_axis_name="s", num_cores=1),
        compiler_params=pltpu.CompilerParams(needs_layout_passes=False),
        scratch_types=[
            pltpu.VMEM((2 * _TBL_PAD,), jnp.float32),
            pltpu.VMEM((_PER_W,), jnp.int32),
            pltpu.VMEM((2 * _PER_W,), jnp.float32),
            pltpu.SemaphoreType.DMA,
            pltpu.SemaphoreType.DMA,
            pltpu.SemaphoreType.DMA,
            pltpu.SemaphoreType.DMA,
        ],
    )


_RF = 1536
_NCHUNK = FEAT // _RF
_NBUF = 3


def _scale_body(coef_hbm, std_hbm, x_hbm, e_hbm, out_hbm, eout_hbm,
                coef_v, std_v, xb, eb, ob, sem_cs, sem_in, sem_out, sem_eout):
    cp_c = pltpu.make_async_copy(coef_hbm, coef_v, sem_cs.at[0])
    cp_s = pltpu.make_async_copy(std_hbm, std_v, sem_cs.at[1])
    cp_c.start()
    cp_s.start()

    def start_in(i, b):
        pltpu.make_async_copy(x_hbm.at[pl.ds(i * _RF, _RF), :], xb.at[b],
                              sem_in.at[b, 0]).start()
        pltpu.make_async_copy(e_hbm.at[pl.ds(i * _RF, _RF), :], eb.at[b],
                              sem_in.at[b, 1]).start()

    for b in range(_NBUF):
        start_in(b, b)
    cp_c.wait()
    cp_s.wait()

    def step(i, _):
        @pl.when(jnp.logical_and(i >= 1, (i - 1 + _NBUF) < _NCHUNK))
        def _():
            bp = lax.rem(i - 1, _NBUF)
            pltpu.make_async_copy(eb.at[bp], eout_hbm.at[pl.ds(0, _RF), :],
                                  sem_eout.at[bp]).wait()
            start_in(i - 1 + _NBUF, bp)

        b = lax.rem(i, _NBUF)
        pltpu.make_async_copy(x_hbm.at[pl.ds(0, _RF), :], xb.at[b],
                              sem_in.at[b, 0]).wait()
        pltpu.make_async_copy(e_hbm.at[pl.ds(0, _RF), :], eb.at[b],
                              sem_in.at[b, 1]).wait()

        @pl.when(i >= _NBUF)
        def _():
            pltpu.make_async_copy(ob.at[b], out_hbm.at[pl.ds(0, _RF), :],
                                  sem_out.at[b]).wait()

        ob[b] = coef_v[...] * xb[b] + std_v[...] * eb[b]
        pltpu.make_async_copy(ob.at[b], out_hbm.at[pl.ds(i * _RF, _RF), :],
                              sem_out.at[b]).start()
        pltpu.make_async_copy(eb.at[b], eout_hbm.at[pl.ds(i * _RF, _RF), :],
                              sem_eout.at[b]).start()
        return 0

    lax.fori_loop(0, _NCHUNK, step, 0)
    for k in range(_NBUF):
        b = (_NCHUNK - _NBUF + k) % _NBUF
        pltpu.make_async_copy(ob.at[b], out_hbm.at[pl.ds(0, _RF), :],
                              sem_out.at[b]).wait()
        pltpu.make_async_copy(eb.at[b], eout_hbm.at[pl.ds(0, _RF), :],
                              sem_eout.at[b]).wait()


def _tc_scale(coef, std, xT, eT):
    return pl.pallas_call(
        _scale_body,
        in_specs=[pl.BlockSpec(memory_space=pl.ANY)] * 4,
        out_specs=(pl.BlockSpec(memory_space=pl.ANY),
                   pl.BlockSpec(memory_space=pl.ANY)),
        out_shape=(jax.ShapeDtypeStruct((FEAT, BATCH), jnp.float32),
                   jax.ShapeDtypeStruct((FEAT, BATCH), jnp.float32)),
        scratch_shapes=[
            pltpu.VMEM((1, BATCH), jnp.float32),
            pltpu.VMEM((1, BATCH), jnp.float32),
            pltpu.VMEM((_NBUF, _RF, BATCH), jnp.float32),
            pltpu.VMEM((_NBUF, _RF, BATCH), jnp.float32),
            pltpu.VMEM((_NBUF, _RF, BATCH), jnp.float32),
            pltpu.SemaphoreType.DMA((2,)),
            pltpu.SemaphoreType.DMA((_NBUF, 2)),
            pltpu.SemaphoreType.DMA((_NBUF,)),
            pltpu.SemaphoreType.DMA((_NBUF,)),
        ],
    )(coef, std, xT, eT)


def kernel(x0, timesteps, eps):
    tbl = _schedule_tables()
    cs = _sc_gather_fn()(timesteps.astype(jnp.int32), tbl)
    coef, std = cs[:BATCH], cs[_TBL_PAD:_TBL_PAD + BATCH]
    xT = x0.transpose(1, 2, 3, 0).reshape(FEAT, BATCH)
    eT = eps.transpose(1, 2, 3, 0).reshape(FEAT, BATCH)
    outT, eoutT = _tc_scale(coef.reshape(1, BATCH), std.reshape(1, BATCH),
                            xT, eT)
    sample = outT.reshape(IMG_SHAPE + (BATCH,)).transpose(3, 0, 1, 2)
    eps_out = eoutT.reshape(IMG_SHAPE + (BATCH,)).transpose(3, 0, 1, 2)
    return (sample, eps_out)


# speedup vs baseline: 1.0558x; 1.0066x over previous
import functools

import jax
import jax.numpy as jnp
from jax import lax
from jax.experimental import pallas as pl
from jax.experimental.pallas import tpu as pltpu
from jax.experimental.pallas import tpu_sc as plsc

NUM_T = 1000
IMG_SHAPE = (3, 64, 64)
BATCH = 1024
FEAT = 3 * 64 * 64

_NC = 1
_NS = 16
_L = 16
_NW = _NC * _NS
_PER_W = BATCH // _NW
_TBL_PAD = 1024


def _schedule_tables():
    scale = 1000.0 / NUM_T
    beta = jnp.linspace(scale * 0.0001, scale * 0.02, NUM_T, dtype=jnp.float32)
    alpha_cum = jnp.cumprod(1.0 - beta, axis=0)
    sqrt_ac = jnp.sqrt(alpha_cum)
    sqrt_omac = jnp.sqrt(1.0 - alpha_cum)
    pad = _TBL_PAD - NUM_T
    return jnp.concatenate(
        [jnp.pad(sqrt_ac, (0, pad)), jnp.pad(sqrt_omac, (0, pad))])


def _sc_gather_body(ts_hbm, tbl_hbm, out_hbm, tbl_v, idx_v, res_v,
                    sem_t, sem_i, sem_c, sem_s):
    wid = lax.axis_index("s") * _NC + lax.axis_index("c")
    base = wid * _PER_W
    cp_t = pltpu.make_async_copy(tbl_hbm, tbl_v, sem_t)
    cp_i = pltpu.make_async_copy(ts_hbm.at[pl.ds(base, _PER_W)], idx_v, sem_i)
    cp_t.start()
    cp_i.start()
    cp_t.wait()
    cp_i.wait()
    for j in range(_PER_W // _L):
        idx = idx_v[pl.ds(j * _L, _L)]
        res_v[pl.ds(j * _L, _L)] = plsc.load_gather(tbl_v, [idx])
        res_v[pl.ds(_PER_W + j * _L, _L)] = plsc.load_gather(
            tbl_v, [idx + _TBL_PAD])
    cp_c = pltpu.make_async_copy(res_v.at[pl.ds(0, _PER_W)],
                                 out_hbm.at[pl.ds(base, _PER_W)], sem_c)
    cp_s = pltpu.make_async_copy(res_v.at[pl.ds(_PER_W, _PER_W)],
                                 out_hbm.at[pl.ds(_TBL_PAD + base, _PER_W)],
                                 sem_s)
    cp_c.start()
    cp_s.start()
    cp_c.wait()
    cp_s.wait()


@functools.lru_cache(maxsize=None)
def _sc_gather_fn():
    return pl.kernel(
        _sc_gather_body,
        out_type=jax.ShapeDtypeStruct((2 * _TBL_PAD,), jnp.float32),
        mesh=plsc.VectorSubcoreMesh(core_axis_name="c", subcore_axis_name="s", num_cores=1),
        compiler_params=pltpu.CompilerParams(needs_layout_passes=False),
        scratch_types=[
            pltpu.VMEM((2 * _TBL_PAD,), jnp.float32),
            pltpu.VMEM((_PER_W,), jnp.int32),
            pltpu.VMEM((2 * _PER_W,), jnp.float32),
            pltpu.SemaphoreType.DMA,
            pltpu.SemaphoreType.DMA,
            pltpu.SemaphoreType.DMA,
            pltpu.SemaphoreType.DMA,
        ],
    )


_RF = 1024
_NCHUNK = FEAT // _RF
_NBUF = 4


def _scale_body(coef_hbm, std_hbm, x_hbm, e_hbm, out_hbm, eout_hbm,
                coef_v, std_v, xb, eb, ob, sem_cs, sem_in, sem_out, sem_eout):
    cp_c = pltpu.make_async_copy(coef_hbm, coef_v, sem_cs.at[0])
    cp_s = pltpu.make_async_copy(std_hbm, std_v, sem_cs.at[1])
    cp_c.start()
    cp_s.start()

    def start_in(i, b):
        pltpu.make_async_copy(x_hbm.at[pl.ds(i * _RF, _RF), :], xb.at[b],
                              sem_in.at[b, 0]).start()
        pltpu.make_async_copy(e_hbm.at[pl.ds(i * _RF, _RF), :], eb.at[b],
                              sem_in.at[b, 1]).start()

    for b in range(_NBUF):
        start_in(b, b)
    cp_c.wait()
    cp_s.wait()

    def step(i, _):
        @pl.when(jnp.logical_and(i >= 1, (i - 1 + _NBUF) < _NCHUNK))
        def _():
            bp = lax.rem(i - 1, _NBUF)
            pltpu.make_async_copy(eb.at[bp], eout_hbm.at[pl.ds(0, _RF), :],
                                  sem_eout.at[bp]).wait()
            start_in(i - 1 + _NBUF, bp)

        b = lax.rem(i, _NBUF)
        pltpu.make_async_copy(x_hbm.at[pl.ds(0, _RF), :], xb.at[b],
                              sem_in.at[b, 0]).wait()
        pltpu.make_async_copy(e_hbm.at[pl.ds(0, _RF), :], eb.at[b],
                              sem_in.at[b, 1]).wait()

        @pl.when(i >= _NBUF)
        def _():
            pltpu.make_async_copy(ob.at[b], out_hbm.at[pl.ds(0, _RF), :],
                                  sem_out.at[b]).wait()

        ob[b] = coef_v[...] * xb[b] + std_v[...] * eb[b]
        pltpu.make_async_copy(ob.at[b], out_hbm.at[pl.ds(i * _RF, _RF), :],
                              sem_out.at[b]).start()
        pltpu.make_async_copy(eb.at[b], eout_hbm.at[pl.ds(i * _RF, _RF), :],
                              sem_eout.at[b]).start()
        return 0

    lax.fori_loop(0, _NCHUNK, step, 0)
    for k in range(_NBUF):
        b = (_NCHUNK - _NBUF + k) % _NBUF
        pltpu.make_async_copy(ob.at[b], out_hbm.at[pl.ds(0, _RF), :],
                              sem_out.at[b]).wait()
        pltpu.make_async_copy(eb.at[b], eout_hbm.at[pl.ds(0, _RF), :],
                              sem_eout.at[b]).wait()


def _tc_scale(coef, std, xT, eT):
    return pl.pallas_call(
        _scale_body,
        in_specs=[pl.BlockSpec(memory_space=pl.ANY)] * 4,
        out_specs=(pl.BlockSpec(memory_space=pl.ANY),
                   pl.BlockSpec(memory_space=pl.ANY)),
        out_shape=(jax.ShapeDtypeStruct((FEAT, BATCH), jnp.float32),
                   jax.ShapeDtypeStruct((FEAT, BATCH), jnp.float32)),
        scratch_shapes=[
            pltpu.VMEM((1, BATCH), jnp.float32),
            pltpu.VMEM((1, BATCH), jnp.float32),
            pltpu.VMEM((_NBUF, _RF, BATCH), jnp.float32),
            pltpu.VMEM((_NBUF, _RF, BATCH), jnp.float32),
            pltpu.VMEM((_NBUF, _RF, BATCH), jnp.float32),
            pltpu.SemaphoreType.DMA((2,)),
            pltpu.SemaphoreType.DMA((_NBUF, 2)),
            pltpu.SemaphoreType.DMA((_NBUF,)),
            pltpu.SemaphoreType.DMA((_NBUF,)),
        ],
    )(coef, std, xT, eT)


def kernel(x0, timesteps, eps):
    tbl = _schedule_tables()
    cs = _sc_gather_fn()(timesteps.astype(jnp.int32), tbl)
    coef, std = cs[:BATCH], cs[_TBL_PAD:_TBL_PAD + BATCH]
    xT = x0.transpose(1, 2, 3, 0).reshape(FEAT, BATCH)
    eT = eps.transpose(1, 2, 3, 0).reshape(FEAT, BATCH)
    outT, eoutT = _tc_scale(coef.reshape(1, BATCH), std.reshape(1, BATCH),
                            xT, eT)
    sample = outT.reshape(IMG_SHAPE + (BATCH,)).transpose(3, 0, 1, 2)
    eps_out = eoutT.reshape(IMG_SHAPE + (BATCH,)).transpose(3, 0, 1, 2)
    return (sample, eps_out)
